# initial kernel scaffold (unmeasured)
import jax
import jax.numpy as jnp
from jax import lax
from jax.experimental import pallas as pl
from jax.experimental.pallas import tpu as pltpu

M_CHUNK = 768


def kernel(A, B):
    m, k = A.shape
    k2, n = B.shape
    assert k == k2
    n_chunks = m // M_CHUNK

    def body(a_ref, b_ref, out_ref, comm_ref, send_sem, recv_sem):
        my_x = lax.axis_index("x")
        my_y = lax.axis_index("y")
        peer = (1 - my_x, my_y)

        barrier_sem = pltpu.get_barrier_semaphore()
        pl.semaphore_signal(
            barrier_sem, inc=1, device_id=peer,
            device_id_type=pl.DeviceIdType.MESH,
        )
        pl.semaphore_wait(barrier_sem, 1)

        for i in range(n_chunks):
            sl = pl.ds(i * M_CHUNK, M_CHUNK)
            out_ref[sl, :] = jnp.dot(
                a_ref[sl, :], b_ref[:, :],
                preferred_element_type=jnp.float32,
            )

        rdma = pltpu.make_async_remote_copy(
            src_ref=out_ref,
            dst_ref=comm_ref,
            send_sem=send_sem,
            recv_sem=recv_sem,
            device_id=peer,
            device_id_type=pl.DeviceIdType.MESH,
        )
        rdma.start()
        rdma.wait()

        out_ref[:, :] += comm_ref[:, :]

    return pl.pallas_call(
        body,
        out_shape=jax.ShapeDtypeStruct((m, n), jnp.float32),
        in_specs=[
            pl.BlockSpec(memory_space=pltpu.VMEM),
            pl.BlockSpec(memory_space=pltpu.VMEM),
        ],
        out_specs=pl.BlockSpec(memory_space=pltpu.VMEM),
        scratch_shapes=[
            pltpu.VMEM((m, n), jnp.float32),
            pltpu.SemaphoreType.DMA,
            pltpu.SemaphoreType.DMA,
        ],
        compiler_params=pltpu.CompilerParams(collective_id=0),
    )(A, B)


# baseline (device time: 514216 ns/iter reference)
import jax
import jax.numpy as jnp
from jax import lax
from jax.experimental import pallas as pl
from jax.experimental.pallas import tpu as pltpu

M_CHUNK = 512


def kernel(A, B):
    m, k = A.shape
    k2, n = B.shape
    assert k == k2
    nc = m // M_CHUNK
    assert nc * M_CHUNK == m

    def body(a_hbm, b_ref, out_hbm, a_vm, p_vm, r_vm, o_vm,
             send_sem, recv_sem, load_sem, store_sem, credit_sem):
        my_x = lax.axis_index("x")
        my_y = lax.axis_index("y")
        peer = (1 - my_x, my_y)

        barrier_sem = pltpu.get_barrier_semaphore()
        pl.semaphore_signal(
            barrier_sem, inc=1, device_id=peer,
            device_id_type=pl.DeviceIdType.MESH,
        )
        pl.semaphore_wait(barrier_sem, 1)

        for i in range(nc):
            sl = pl.ds(i * M_CHUNK, M_CHUNK)

            load = pltpu.make_async_copy(a_hbm.at[sl, :], a_vm, load_sem)
            load.start()
            load.wait()

            p_vm[...] = jnp.dot(
                a_vm[...], b_ref[...], preferred_element_type=jnp.float32
            )

            if i >= 1:
                pl.semaphore_wait(credit_sem, 1)

            rdma = pltpu.make_async_remote_copy(
                src_ref=p_vm,
                dst_ref=r_vm,
                send_sem=send_sem,
                recv_sem=recv_sem,
                device_id=peer,
                device_id_type=pl.DeviceIdType.MESH,
            )
            rdma.start()
            rdma.wait()

            o_vm[...] = p_vm[...] + r_vm[...]

            if i < nc - 1:
                pl.semaphore_signal(
                    credit_sem, inc=1, device_id=peer,
                    device_id_type=pl.DeviceIdType.MESH,
                )

            store = pltpu.make_async_copy(o_vm, out_hbm.at[sl, :], store_sem)
            store.start()
            store.wait()

    return pl.pallas_call(
        body,
        out_shape=jax.ShapeDtypeStruct((m, n), jnp.float32),
        in_specs=[
            pl.BlockSpec(memory_space=pl.ANY),
            pl.BlockSpec(memory_space=pltpu.VMEM),
        ],
        out_specs=pl.BlockSpec(memory_space=pl.ANY),
        scratch_shapes=[
            pltpu.VMEM((M_CHUNK, k), jnp.float32),
            pltpu.VMEM((M_CHUNK, n), jnp.float32),
            pltpu.VMEM((M_CHUNK, n), jnp.float32),
            pltpu.VMEM((M_CHUNK, n), jnp.float32),
            pltpu.SemaphoreType.DMA,
            pltpu.SemaphoreType.DMA,
            pltpu.SemaphoreType.DMA,
            pltpu.SemaphoreType.DMA,
            pltpu.SemaphoreType.REGULAR,
        ],
        compiler_params=pltpu.CompilerParams(
            collective_id=0, vmem_limit_bytes=60 * 1024 * 1024
        ),
    )(A, B)


# device time: 248837 ns/iter; 2.0665x vs baseline; 2.0665x over previous
import jax
import jax.numpy as jnp
from jax import lax
from jax.experimental import pallas as pl
from jax.experimental.pallas import tpu as pltpu

M_CHUNK = 512


def kernel(A, B):
    m, k = A.shape
    k2, n = B.shape
    assert k == k2
    nc = m // M_CHUNK
    assert nc * M_CHUNK == m and nc >= 3

    def body(a_hbm, b_ref, out_hbm, a_vm, p_vm, s_vm, r_vm,
             send_sems, recv_sems, load_sems, store_sems, credit_sem):
        my_x = lax.axis_index("x")
        my_y = lax.axis_index("y")
        peer = (1 - my_x, my_y)

        barrier_sem = pltpu.get_barrier_semaphore()
        pl.semaphore_signal(
            barrier_sem, inc=1, device_id=peer,
            device_id_type=pl.DeviceIdType.MESH,
        )
        pl.semaphore_wait(barrier_sem, 1)

        def chunk(i):
            return pl.ds(i * M_CHUNK, M_CHUNK)

        def load(i):
            return pltpu.make_async_copy(
                a_hbm.at[chunk(i), :], a_vm.at[i % 2], load_sems.at[i % 2]
            )

        def store(i):
            return pltpu.make_async_copy(
                p_vm.at[i % 2], out_hbm.at[chunk(i), :], store_sems.at[i % 2]
            )

        def rdma(i):
            return pltpu.make_async_remote_copy(
                src_ref=s_vm.at[i % 2],
                dst_ref=r_vm.at[i % 2],
                send_sem=send_sems.at[i % 2],
                recv_sem=recv_sems.at[i % 2],
                device_id=peer,
                device_id_type=pl.DeviceIdType.MESH,
            )

        def compute(i):
            slot = i % 2
            p_vm[slot] = jnp.dot(
                a_vm[slot], b_ref[...], preferred_element_type=jnp.float32
            )

        load(0).start()
        load(1).start()
        load(0).wait()
        compute(0)
        s_vm[0] = p_vm[0].astype(jnp.bfloat16)
        rdma(0).start()

        for i in range(nc):
            slot = i % 2
            nslot = (i + 1) % 2

            if i + 1 < nc:
                if i + 2 < nc:
                    load(i + 2).start()
                load(i + 1).wait()
                if i >= 1:
                    store(i - 1).wait()
                compute(i + 1)
                if i >= 1:
                    rdma(i - 1).wait_send()
                s_vm[nslot] = p_vm[nslot].astype(jnp.bfloat16)
                if i + 1 >= 2:
                    pl.semaphore_wait(credit_sem, 1)
                rdma(i + 1).start()

            rdma(i).wait_recv()
            p_vm[slot] = p_vm[slot] + r_vm[slot].astype(jnp.float32)
            if i <= nc - 3:
                pl.semaphore_signal(
                    credit_sem, inc=1, device_id=peer,
                    device_id_type=pl.DeviceIdType.MESH,
                )
            store(i).start()

        rdma(nc - 2).wait_send()
        rdma(nc - 1).wait_send()
        store(nc - 2).wait()
        store(nc - 1).wait()

    return pl.pallas_call(
        body,
        out_shape=jax.ShapeDtypeStruct((m, n), jnp.float32),
        in_specs=[
            pl.BlockSpec(memory_space=pl.ANY),
            pl.BlockSpec(memory_space=pltpu.VMEM),
        ],
        out_specs=pl.BlockSpec(memory_space=pl.ANY),
        scratch_shapes=[
            pltpu.VMEM((2, M_CHUNK, k), jnp.float32),
            pltpu.VMEM((2, M_CHUNK, n), jnp.float32),
            pltpu.VMEM((2, M_CHUNK, n), jnp.bfloat16),
            pltpu.VMEM((2, M_CHUNK, n), jnp.bfloat16),
            pltpu.SemaphoreType.DMA((2,)),
            pltpu.SemaphoreType.DMA((2,)),
            pltpu.SemaphoreType.DMA((2,)),
            pltpu.SemaphoreType.DMA((2,)),
            pltpu.SemaphoreType.REGULAR,
        ],
        compiler_params=pltpu.CompilerParams(
            collective_id=0, vmem_limit_bytes=60 * 1024 * 1024
        ),
    )(A, B)
